# Initial kernel scaffold; baseline (speedup 1.0000x reference)
import jax
import jax.numpy as jnp
from jax.experimental import pallas as pl


def kernel(seq_len, table):
    raise NotImplementedError("write your pallas kernel here")


# speedup vs baseline: 1.5514x; 1.5514x over previous
import functools

import jax
import jax.numpy as jnp
from jax import lax
from jax.experimental import pallas as pl
from jax.experimental.pallas import tpu as pltpu
from jax.experimental.pallas import tpu_sc as plsc

MAX_ROWS = 8192
EMB = 1024
NC = 2
NS = 16
NW = NC * NS
B_PER_W = MAX_ROWS // NW
CHUNK = 64
N_CHUNKS = B_PER_W // CHUNK


def _gather_body(table_hbm, idx_hbm, out_hbm, idx_v, buf_v, sem):
    wid = lax.axis_index("s") * NC + lax.axis_index("c")
    base = wid * B_PER_W
    pltpu.sync_copy(idx_hbm.at[pl.ds(base, B_PER_W)], idx_v)

    def chunk(g, _):
        pltpu.async_copy(
            table_hbm.at[idx_v.at[pl.ds(g * CHUNK, CHUNK)]], buf_v, sem
        ).wait()
        pltpu.sync_copy(buf_v, out_hbm.at[pl.ds(base + g * CHUNK, CHUNK)])
        return ()

    lax.fori_loop(0, N_CHUNKS, chunk, (), unroll=False)


_sc_gather = functools.partial(
    pl.kernel,
    out_type=jax.ShapeDtypeStruct((MAX_ROWS, EMB), jnp.float32),
    mesh=plsc.VectorSubcoreMesh(core_axis_name="c", subcore_axis_name="s"),
    scratch_types=[
        pltpu.VMEM((B_PER_W,), jnp.int32),
        pltpu.VMEM((CHUNK, EMB), jnp.float32),
        pltpu.SemaphoreType.DMA,
    ],
)(_gather_body)


def kernel(seq_len, table):
    n = table.shape[0]
    offset = jnp.asarray(seq_len, dtype=jnp.int32) - jnp.int32(n)
    idx = jnp.clip(jnp.arange(n, dtype=jnp.int32) + offset, 0, n - 1)
    return _sc_gather(table, idx)
